# Initial kernel scaffold; baseline (speedup 1.0000x reference)
import jax
import jax.numpy as jnp
from jax.experimental import pallas as pl


def kernel(source_features, target_features):
    raise NotImplementedError("write your pallas kernel here")


# speedup vs baseline: 6.3014x; 6.3014x over previous
import functools

import jax
import jax.numpy as jnp
from jax import lax
from jax.experimental import pallas as pl
from jax.experimental.pallas import tpu as pltpu
from jax.experimental.pallas import tpu_sc as plsc

Q = 8192
T = 8192
D = 1024
BQ = 2048
BT = 1024
NQ = Q // BQ
NT = T // BT


def _argmin_body(s_ref, t_ref, ns_ref, nt_ref, idx_ref, bd_ref, bi_ref):
    j = pl.program_id(1)

    @pl.when(j == 0)
    def _init():
        bd_ref[...] = jnp.full((BQ, 1), jnp.inf, jnp.float32)
        bi_ref[...] = jnp.zeros((BQ, 1), jnp.int32)

    dot = lax.dot_general(
        s_ref[...], t_ref[...], (((1,), (1,)), ((), ())),
        preferred_element_type=jnp.float32)
    d = 1.0 - dot / (ns_ref[...] * nt_ref[...])
    m = jnp.min(d, axis=1, keepdims=True)
    iota = lax.broadcasted_iota(jnp.int32, (BQ, BT), 1)
    a = jnp.min(jnp.where(d == m, iota, BT), axis=1, keepdims=True)
    gi = j * BT + a
    upd = m < bd_ref[...]
    bd_ref[...] = jnp.where(upd, m, bd_ref[...])
    bi_ref[...] = jnp.where(upd, gi, bi_ref[...])

    @pl.when(j == NT - 1)
    def _emit():
        idx_ref[...] = bi_ref[...]


def _argmin_indices(source, target, ns_col, nt_row, interpret=False):
    return pl.pallas_call(
        _argmin_body,
        grid=(NQ, NT),
        in_specs=[
            pl.BlockSpec((BQ, D), lambda i, j: (i, 0)),
            pl.BlockSpec((BT, D), lambda i, j: (j, 0)),
            pl.BlockSpec((BQ, 1), lambda i, j: (i, 0)),
            pl.BlockSpec((1, BT), lambda i, j: (0, j)),
        ],
        out_specs=pl.BlockSpec((BQ, 1), lambda i, j: (i, 0)),
        out_shape=jax.ShapeDtypeStruct((Q, 1), jnp.int32),
        scratch_shapes=[
            pltpu.VMEM((BQ, 1), jnp.float32),
            pltpu.VMEM((BQ, 1), jnp.int32),
        ],
        compiler_params=pltpu.CompilerParams(
            dimension_semantics=("arbitrary", "arbitrary")),
        interpret=interpret,
    )(source, target, ns_col, nt_row)


_NC = 2
_NS = 16
_NW = _NC * _NS
_BPW = Q // _NW
_CH = 64
_NCH = _BPW // _CH


def _gather_body(table_hbm, idx_hbm, out_hbm, idx_v, rows_v, sem):
    wid = lax.axis_index("s") * _NC + lax.axis_index("c")
    base = wid * _BPW
    for c in range(_NCH):
        off = base + c * _CH
        pltpu.sync_copy(idx_hbm.at[pl.ds(off, _CH)], idx_v)
        pltpu.async_copy(table_hbm.at[idx_v], rows_v, sem).wait()
        pltpu.sync_copy(rows_v, out_hbm.at[pl.ds(off, _CH)])


def _sc_gather(table, idx):
    k = functools.partial(
        pl.kernel,
        mesh=plsc.VectorSubcoreMesh(
            core_axis_name="c", subcore_axis_name="s",
            num_cores=_NC, num_subcores=_NS),
        out_type=jax.ShapeDtypeStruct((Q, D), jnp.float32),
        scratch_types=[
            pltpu.VMEM((_CH,), jnp.int32),
            pltpu.VMEM((_CH, D), jnp.float32),
            pltpu.SemaphoreType.DMA,
        ],
    )(_gather_body)
    return k(table, idx)


def kernel(source_features, target_features):
    source_norms = jnp.linalg.norm(source_features, axis=-1)
    matching_norms = jnp.linalg.norm(target_features, axis=-1)
    idx = _argmin_indices(
        source_features, target_features,
        source_norms.reshape(Q, 1), matching_norms.reshape(1, T))
    idx = idx.reshape(Q)
    return _sc_gather(target_features, idx)


# speedup vs baseline: 6.6141x; 1.0496x over previous
import functools

import jax
import jax.numpy as jnp
from jax import lax
from jax.experimental import pallas as pl
from jax.experimental.pallas import tpu as pltpu
from jax.experimental.pallas import tpu_sc as plsc

Q = 8192
T = 8192
D = 1024
BQ = 2048
BT = 512
NQ = Q // BQ
NT = T // BT
S = NT // 4
CR = 256
NCR = BQ // CR

_DN = (((1,), (1,)), ((), ()))


def _phase(s_ref, bd_ref, bi_ref, dsts, t_refs, srcs, nsp_ref, nt_refs,
           col_blocks, valid):
    for r in range(NCR):
        sl = pl.ds(r * CR, CR)
        for dst, t_ref in zip(dsts, t_refs):
            dst[sl, :] = lax.dot_general(
                s_ref[sl, :], t_ref[...], _DN,
                preferred_element_type=jnp.float32)
        for src, nt_ref, cb in zip(srcs, nt_refs, col_blocks):
            d = 1.0 - src[sl, :] / (nsp_ref[sl, :] * nt_ref[...])
            m = jnp.min(d, axis=1, keepdims=True)
            iota = lax.broadcasted_iota(jnp.int32, (CR, BT), 1)
            a = jnp.min(jnp.where(d == m, iota, BT), axis=1, keepdims=True)
            gi = cb * BT + a
            upd = jnp.logical_and(m < bd_ref[sl, :], valid)
            bd_ref[sl, :] = jnp.where(upd, m, bd_ref[sl, :])
            bi_ref[sl, :] = jnp.where(upd, gi, bi_ref[sl, :])


def _argmin_body(s_ref, ta_ref, tb_ref, tc_ref, td_ref,
                 nsp_ref, nsq_ref, ntc_ref, ntd_ref, nta_ref, ntb_ref,
                 idx_ref, a_buf, b_buf, c_buf, d_buf, bd_ref, bi_ref):
    g = pl.program_id(0)
    q = g % S
    pe = (g - 1) % S
    glast = pl.num_programs(0) - 1

    _phase(s_ref, bd_ref, bi_ref,
           (a_buf, b_buf), (ta_ref, tb_ref),
           (c_buf, d_buf), nsp_ref, (ntc_ref, ntd_ref),
           (4 * pe + 2, 4 * pe + 3), g > 0)

    @pl.when(jnp.logical_and(g > 0, q == 0))
    def _emit():
        idx_ref[...] = bi_ref[...]

    @pl.when(q == 0)
    def _init():
        bd_ref[...] = jnp.full((BQ, 1), jnp.inf, jnp.float32)
        bi_ref[...] = jnp.zeros((BQ, 1), jnp.int32)

    _phase(s_ref, bd_ref, bi_ref,
           (c_buf, d_buf), (tc_ref, td_ref),
           (a_buf, b_buf), nsq_ref, (nta_ref, ntb_ref),
           (4 * q, 4 * q + 1), g < glast)


def _argmin_indices(source, target, ns_col, nt_row, interpret=False):
    grid = (NQ * S + 1,)
    return pl.pallas_call(
        _argmin_body,
        grid=grid,
        in_specs=[
            pl.BlockSpec((BQ, D), lambda g: (jnp.minimum(g // S, NQ - 1), 0)),
            pl.BlockSpec((BT, D), lambda g: (4 * (g % S), 0)),
            pl.BlockSpec((BT, D), lambda g: (4 * (g % S) + 1, 0)),
            pl.BlockSpec((BT, D), lambda g: (4 * (g % S) + 2, 0)),
            pl.BlockSpec((BT, D), lambda g: (4 * (g % S) + 3, 0)),
            pl.BlockSpec((BQ, 1), lambda g: (jnp.maximum((g - 1) // S, 0), 0)),
            pl.BlockSpec((BQ, 1), lambda g: (jnp.minimum(g // S, NQ - 1), 0)),
            pl.BlockSpec((1, BT), lambda g: (0, 4 * ((g - 1) % S) + 2)),
            pl.BlockSpec((1, BT), lambda g: (0, 4 * ((g - 1) % S) + 3)),
            pl.BlockSpec((1, BT), lambda g: (0, 4 * (g % S))),
            pl.BlockSpec((1, BT), lambda g: (0, 4 * (g % S) + 1)),
        ],
        out_specs=pl.BlockSpec((BQ, 1), lambda g: (jnp.maximum(g // S - 1, 0), 0)),
        out_shape=jax.ShapeDtypeStruct((Q, 1), jnp.int32),
        scratch_shapes=[
            pltpu.VMEM((BQ, BT), jnp.float32),
            pltpu.VMEM((BQ, BT), jnp.float32),
            pltpu.VMEM((BQ, BT), jnp.float32),
            pltpu.VMEM((BQ, BT), jnp.float32),
            pltpu.VMEM((BQ, 1), jnp.float32),
            pltpu.VMEM((BQ, 1), jnp.int32),
        ],
        compiler_params=pltpu.CompilerParams(
            dimension_semantics=("arbitrary",)),
        interpret=interpret,
    )(source, target, target, target, target,
      ns_col, ns_col, nt_row, nt_row, nt_row, nt_row)


_NC = 2
_NS = 16
_NW = _NC * _NS
_BPW = Q // _NW
_CH = 64
_NCH = _BPW // _CH


def _gather_body(table_hbm, idx_hbm, out_hbm, idx_v, rows_v, sem):
    wid = lax.axis_index("s") * _NC + lax.axis_index("c")
    base = wid * _BPW
    for c in range(_NCH):
        off = base + c * _CH
        pltpu.sync_copy(idx_hbm.at[pl.ds(off, _CH)], idx_v)
        pltpu.async_copy(table_hbm.at[idx_v], rows_v, sem).wait()
        pltpu.sync_copy(rows_v, out_hbm.at[pl.ds(off, _CH)])


def _sc_gather(table, idx):
    k = functools.partial(
        pl.kernel,
        mesh=plsc.VectorSubcoreMesh(
            core_axis_name="c", subcore_axis_name="s",
            num_cores=_NC, num_subcores=_NS),
        out_type=jax.ShapeDtypeStruct((Q, D), jnp.float32),
        scratch_types=[
            pltpu.VMEM((_CH,), jnp.int32),
            pltpu.VMEM((_CH, D), jnp.float32),
            pltpu.SemaphoreType.DMA,
        ],
    )(_gather_body)
    return k(table, idx)


def kernel(source_features, target_features):
    source_norms = jnp.linalg.norm(source_features, axis=-1)
    matching_norms = jnp.linalg.norm(target_features, axis=-1)
    idx = _argmin_indices(
        source_features, target_features,
        source_norms.reshape(Q, 1), matching_norms.reshape(1, T))
    idx = idx.reshape(Q)
    return _sc_gather(target_features, idx)
